# baseline (speedup 1.0000x reference)
import functools

import jax
import jax.numpy as jnp
from jax.experimental import pallas as pl
from jax.experimental.pallas import tpu as pltpu

N = 65536
D = 256
K = 8
PROWS = N // K
BLK = 8192
NB = N // BLK
CH = BLK // K
RROWS = N // 128
PAVG = PROWS // 128
TAVG = CH // 128
K_FORCE = max(1, int(N * 0.1))
K_CAP = int(N * 0.5)


def _body(*refs):
    thr_ref = refs[0]
    u0_ref, u1_ref, u2_ref = refs[1:4]
    lab_refs = refs[4:4 + K]
    e_refs = refs[4 + K:4 + 2 * K]
    ms_ref = refs[4 + 2 * K]
    o_refs = refs[5 + 2 * K:5 + 3 * K]
    avg_s, acc_s, cnt_s, cen_s, c2_s = refs[5 + 3 * K:]

    p = pl.program_id(0)
    j = pl.program_id(1)

    @pl.when((p == 0) & (j == 0))
    def _init():
        acc_s[...] = jnp.zeros_like(acc_s)
        cnt_s[...] = jnp.zeros_like(cnt_s)

    @pl.when(p == 0)
    def _phase0():
        for i in range(K):
            sl = pl.ds(i * PAVG + j * TAVG, TAVG)
            avg_s[sl, :] = (u0_ref[sl, :] + u1_ref[sl, :] + u2_ref[sl, :]) / 3.0

            e = e_refs[i][0]
            lab = lab_refs[i][0]
            m0 = (lab == 0).astype(jnp.float32)
            m1 = (lab == 1).astype(jnp.float32)
            st = jnp.sum(e, axis=0, keepdims=True)
            s0 = jnp.sum(e * m0, axis=0, keepdims=True)
            s1 = jnp.sum(e * m1, axis=0, keepdims=True)
            acc_s[0:1, :] += s0
            acc_s[1:2, :] += s1
            acc_s[2:3, :] += st - s0 - s1
            cnt_s[0:1, 0:1] += jnp.sum(m0, axis=(0, 1), keepdims=True)
            cnt_s[0:1, 1:2] += jnp.sum(m1, axis=(0, 1), keepdims=True)

    @pl.when((p == 0) & (j == NB - 1))
    def _finalize():
        cnt_row = jnp.concatenate(
            [cnt_s[0:1, 0:2],
             jnp.full((1, 1), float(N), jnp.float32)
             - cnt_s[0:1, 0:1] - cnt_s[0:1, 1:2],
             jnp.zeros((1, 5), jnp.float32)], axis=1)
        invc = 1.0 / jnp.maximum(cnt_row, 1.0)
        cen = jnp.transpose(acc_s[...]) * invc
        cen_s[...] = cen
        c2_s[...] = jnp.sum(cen * cen, axis=0, keepdims=True)

        avg = avg_s[...]
        thr = thr_ref[0]
        keys = jax.lax.bitcast_convert_type(avg, jnp.int32)
        hard_count = jnp.sum((avg > thr).astype(jnp.int32))
        k_eff = jnp.where(hard_count == 0, K_FORCE, K_CAP)

        def bs(_, carry):
            lo, hi = carry
            mid = jax.lax.shift_right_arithmetic(lo + hi, 1)
            cnt = jnp.sum((keys > mid).astype(jnp.int32))
            pred = cnt >= k_eff
            return (jnp.where(pred, mid, lo), jnp.where(pred, hi, mid))

        lo, hi = jax.lax.fori_loop(
            0, 31, bs, (jnp.int32(-1), jnp.int32(0x40000000)))
        use_thr = (hard_count > 0) & (hard_count <= K_CAP)
        sel = jnp.where(use_thr, (avg > thr).astype(jnp.float32),
                        (keys >= hi).astype(jnp.float32))
        ms_ref[...] = avg * sel

    @pl.when(p == 1)
    def _phase1():
        c2 = c2_s[...]
        cen = cen_s[...]
        for i in range(K):
            e = e_refs[i][0]
            prod = jax.lax.dot_general(
                e, cen, (((1,), (0,)), ((), ())),
                preferred_element_type=jnp.float32)
            x2 = jnp.sum(e * e, axis=1, keepdims=True)
            d2 = jnp.maximum(x2 + c2 - 2.0 * prod, 0.0)
            o_refs[i][...] = jnp.sqrt(d2 + 1e-12)


def _specs():
    e_specs = [
        pl.BlockSpec((1, CH, D), (lambda p, j, i=i: (i, j, 0)))
        for i in range(K)
    ]
    lab_specs = [
        pl.BlockSpec((1, CH, 1),
                     (lambda p, j, i=i: (i, jnp.where(p == 0, j, 0), 0)))
        for i in range(K)
    ]
    o_specs = [
        pl.BlockSpec((CH, 8), lambda p, j: (jnp.where(p == 0, 0, j), 0))
        for _ in range(K)
    ]
    return dict(
        grid=(2, NB),
        in_specs=(
            [pl.BlockSpec(memory_space=pltpu.SMEM)]
            + [pl.BlockSpec((RROWS, 128), lambda p, j: (0, 0))] * 3
            + lab_specs + e_specs
        ),
        out_specs=[pl.BlockSpec((RROWS, 128), lambda p, j: (0, 0))] + o_specs,
        out_shape=(
            [jax.ShapeDtypeStruct((RROWS, 128), jnp.float32)]
            + [jax.ShapeDtypeStruct((PROWS, 8), jnp.float32)] * K
        ),
        scratch_shapes=[
            pltpu.VMEM((RROWS, 128), jnp.float32),
            pltpu.VMEM((8, D), jnp.float32),
            pltpu.VMEM((1, 8), jnp.float32),
            pltpu.VMEM((D, 8), jnp.float32),
            pltpu.VMEM((1, 8), jnp.float32),
        ],
        compiler_params=pltpu.CompilerParams(
            dimension_semantics=("arbitrary", "arbitrary")),
    )


@functools.partial(jax.jit, static_argnames=("interpret",))
def _run(thr, u0, u1, u2, lab8, e8, interpret=False):
    outs = pl.pallas_call(
        _body, **_specs(), interpret=interpret,
    )(thr, u0, u1, u2, *([lab8] * K), *([e8] * K))
    return outs[0], outs[1:]


def kernel(uncertainty, embeddings, labels, epoch, max_epochs):
    progress = jnp.minimum(epoch / jnp.maximum(max_epochs - 1, 1), 1.0)
    u_thr = 0.4 + progress * (0.3 - 0.4)
    c_thr = 0.3 + progress * (0.6 - 0.3)
    thr = jnp.minimum(u_thr, -jnp.log(c_thr)).astype(jnp.float32)
    thr = jnp.reshape(thr, (1,))

    u0 = uncertainty[:, 0].reshape(RROWS, 128)
    u1 = uncertainty[:, 1].reshape(RROWS, 128)
    u2 = uncertainty[:, 2].reshape(RROWS, 128)
    lab8 = labels.reshape(K, PROWS, 1)
    e8 = embeddings.reshape(K, PROWS, D)

    ms, dists = _run(thr, u0, u1, u2, lab8, e8)
    dist = jnp.concatenate(list(dists), axis=0)
    return jnp.concatenate([ms.reshape(N, 1), dist[:, :3]], axis=1)


# speedup vs baseline: 1.0193x; 1.0193x over previous
import functools

import jax
import jax.numpy as jnp
from jax.experimental import pallas as pl
from jax.experimental.pallas import tpu as pltpu

N = 65536
D = 256
K = 8
PROWS = N // K
BLK = 8192
NB = N // BLK
CH = BLK // K
RROWS = N // 128
PAVG = PROWS // 128
TAVG = CH // 128
K_FORCE = max(1, int(N * 0.1))
K_CAP = int(N * 0.5)


def _body(*refs):
    thr_ref = refs[0]
    u0_ref, u1_ref, u2_ref = refs[1:4]
    lab_refs = refs[4:4 + K]
    e_refs = refs[4 + K:4 + 2 * K]
    ms_ref = refs[4 + 2 * K]
    o_refs = refs[5 + 2 * K:5 + 3 * K]
    avg_s, acc_s, cnt_s, cen_s, c2_s = refs[5 + 3 * K:]

    p = pl.program_id(0)
    j = pl.program_id(1)

    @pl.when((p == 0) & (j == 0))
    def _init():
        acc_s[...] = jnp.zeros_like(acc_s)
        cnt_s[...] = jnp.zeros_like(cnt_s)

    @pl.when(p == 0)
    def _phase0():
        for i in range(K):
            sl = pl.ds(i * PAVG + j * TAVG, TAVG)
            avg_s[sl, :] = (u0_ref[sl, :] + u1_ref[sl, :] + u2_ref[sl, :]) / 3.0

            e = e_refs[i][0]
            lf = lab_refs[i][0].astype(jnp.float32)
            le = e * lf
            t0 = jnp.sum(e, axis=0, keepdims=True)
            t1 = jnp.sum(le, axis=0, keepdims=True)
            t2 = jnp.sum(le * lf, axis=0, keepdims=True)
            acc_s[0:1, :] += t0 - t1 + (t2 - t1) * 0.5
            acc_s[1:2, :] += 2.0 * t1 - t2
            acc_s[2:3, :] += (t2 - t1) * 0.5
            cnt_s[0:1, 0:1] += jnp.sum(lf, axis=(0, 1), keepdims=True)
            cnt_s[0:1, 1:2] += jnp.sum(lf * lf, axis=(0, 1), keepdims=True)

    @pl.when((p == 0) & (j == NB - 1))
    def _finalize():
        C1 = cnt_s[0:1, 0:1]
        C2 = cnt_s[0:1, 1:2]
        n2 = (C2 - C1) * 0.5
        n1 = 2.0 * C1 - C2
        n0 = jnp.full((1, 1), float(N), jnp.float32) - n1 - n2
        cnt_row = jnp.concatenate(
            [n0, n1, n2, jnp.zeros((1, 5), jnp.float32)], axis=1)
        invc = 1.0 / jnp.maximum(cnt_row, 1.0)
        cen = jnp.transpose(acc_s[...]) * invc
        cen_s[...] = cen
        c2_s[...] = jnp.sum(cen * cen, axis=0, keepdims=True)

        avg = avg_s[...]
        thr = thr_ref[0]
        keys = jax.lax.bitcast_convert_type(avg, jnp.int32)
        hard_count = jnp.sum((avg > thr).astype(jnp.int32))
        k_eff = jnp.where(hard_count == 0, K_FORCE, K_CAP)

        def bs(_, carry):
            lo, hi = carry
            mid = jax.lax.shift_right_arithmetic(lo + hi, 1)
            cnt = jnp.sum((keys > mid).astype(jnp.int32))
            pred = cnt >= k_eff
            return (jnp.where(pred, mid, lo), jnp.where(pred, hi, mid))

        lo, hi = jax.lax.fori_loop(
            0, 31, bs, (jnp.int32(-1), jnp.int32(0x40000000)))
        use_thr = (hard_count > 0) & (hard_count <= K_CAP)
        sel = jnp.where(use_thr, (avg > thr).astype(jnp.float32),
                        (keys >= hi).astype(jnp.float32))
        ms_ref[...] = avg * sel

    @pl.when(p == 1)
    def _phase1():
        c2 = c2_s[...]
        cen = cen_s[...]
        for i in range(K):
            e = e_refs[i][0]
            prod = jax.lax.dot_general(
                e, cen, (((1,), (0,)), ((), ())),
                preferred_element_type=jnp.float32)
            x2 = jnp.sum(e * e, axis=1, keepdims=True)
            d2 = jnp.maximum(x2 + c2 - 2.0 * prod, 0.0) + 1e-12
            o_refs[i][...] = d2 * jax.lax.rsqrt(d2)


def _specs():
    e_specs = [
        pl.BlockSpec((1, CH, D), (lambda p, j, i=i: (i, j, 0)))
        for i in range(K)
    ]
    lab_specs = [
        pl.BlockSpec((1, CH, 1),
                     (lambda p, j, i=i: (i, jnp.where(p == 0, j, 0), 0)))
        for i in range(K)
    ]
    o_specs = [
        pl.BlockSpec((CH, 8), lambda p, j: (jnp.where(p == 0, 0, j), 0))
        for _ in range(K)
    ]
    return dict(
        grid=(2, NB),
        in_specs=(
            [pl.BlockSpec(memory_space=pltpu.SMEM)]
            + [pl.BlockSpec((RROWS, 128), lambda p, j: (0, 0))] * 3
            + lab_specs + e_specs
        ),
        out_specs=[pl.BlockSpec((RROWS, 128), lambda p, j: (0, 0))] + o_specs,
        out_shape=(
            [jax.ShapeDtypeStruct((RROWS, 128), jnp.float32)]
            + [jax.ShapeDtypeStruct((PROWS, 8), jnp.float32)] * K
        ),
        scratch_shapes=[
            pltpu.VMEM((RROWS, 128), jnp.float32),
            pltpu.VMEM((8, D), jnp.float32),
            pltpu.VMEM((1, 8), jnp.float32),
            pltpu.VMEM((D, 8), jnp.float32),
            pltpu.VMEM((1, 8), jnp.float32),
        ],
        compiler_params=pltpu.CompilerParams(
            dimension_semantics=("arbitrary", "arbitrary")),
    )


@functools.partial(jax.jit, static_argnames=("interpret",))
def _run(thr, u0, u1, u2, lab8, e8, interpret=False):
    outs = pl.pallas_call(
        _body, **_specs(), interpret=interpret,
    )(thr, u0, u1, u2, *([lab8] * K), *([e8] * K))
    return outs[0], outs[1:]


def kernel(uncertainty, embeddings, labels, epoch, max_epochs):
    progress = jnp.minimum(epoch / jnp.maximum(max_epochs - 1, 1), 1.0)
    u_thr = 0.4 + progress * (0.3 - 0.4)
    c_thr = 0.3 + progress * (0.6 - 0.3)
    thr = jnp.minimum(u_thr, -jnp.log(c_thr)).astype(jnp.float32)
    thr = jnp.reshape(thr, (1,))

    u0 = uncertainty[:, 0].reshape(RROWS, 128)
    u1 = uncertainty[:, 1].reshape(RROWS, 128)
    u2 = uncertainty[:, 2].reshape(RROWS, 128)
    lab8 = labels.reshape(K, PROWS, 1)
    e8 = embeddings.reshape(K, PROWS, D)

    ms, dists = _run(thr, u0, u1, u2, lab8, e8)
    dist = jnp.concatenate(list(dists), axis=0)
    return jnp.concatenate([ms.reshape(N, 1), dist[:, :3]], axis=1)


# speedup vs baseline: 1.0703x; 1.0501x over previous
import functools

import jax
import jax.numpy as jnp
from jax.experimental import pallas as pl
from jax.experimental.pallas import tpu as pltpu

N = 65536
D = 256
K = 8
PROWS = N // K
BLK = 8192
NB = N // BLK
CH = BLK // K
RROWS = N // 128
PAVG = PROWS // 128
TAVG = CH // 128
K_FORCE = max(1, int(N * 0.1))
K_CAP = int(N * 0.5)


def _body(*refs):
    thr_ref = refs[0]
    u0_ref, u1_ref, u2_ref = refs[1:4]
    lab_ref = refs[4]
    e_refs = refs[5:5 + K]
    ms_ref = refs[5 + K]
    o_ref = refs[6 + K]
    avg_s, acc_s, cnt_s, cen_s, c2_s = refs[7 + K:]

    p = pl.program_id(0)
    j = pl.program_id(1)

    @pl.when((p == 0) & (j == 0))
    def _init():
        acc_s[...] = jnp.zeros_like(acc_s)
        cnt_s[...] = jnp.zeros_like(cnt_s)

    @pl.when(p == 0)
    def _phase0():
        for i in range(K):
            sl = pl.ds(i * PAVG + j * TAVG, TAVG)
            avg_s[sl, :] = (u0_ref[sl, :] + u1_ref[sl, :] + u2_ref[sl, :]) / 3.0

            e = e_refs[i][0]
            lf = lab_ref[i, 0].astype(jnp.float32)
            le = e * lf
            t0 = jnp.sum(e, axis=0, keepdims=True)
            t1 = jnp.sum(le, axis=0, keepdims=True)
            t2 = jnp.sum(le * lf, axis=0, keepdims=True)
            acc_s[0:1, :] += t0 - t1 + (t2 - t1) * 0.5
            acc_s[1:2, :] += 2.0 * t1 - t2
            acc_s[2:3, :] += (t2 - t1) * 0.5
            cnt_s[0:1, 0:1] += jnp.sum(lf, axis=(0, 1), keepdims=True)
            cnt_s[0:1, 1:2] += jnp.sum(lf * lf, axis=(0, 1), keepdims=True)

    @pl.when((p == 0) & (j == NB - 1))
    def _finalize():
        C1 = cnt_s[0:1, 0:1]
        C2 = cnt_s[0:1, 1:2]
        n2 = (C2 - C1) * 0.5
        n1 = 2.0 * C1 - C2
        n0 = jnp.full((1, 1), float(N), jnp.float32) - n1 - n2
        cnt_row = jnp.concatenate(
            [n0, n1, n2, jnp.zeros((1, 5), jnp.float32)], axis=1)
        invc = 1.0 / jnp.maximum(cnt_row, 1.0)
        cen = jnp.transpose(acc_s[...]) * invc
        cen_s[...] = cen
        c2_s[...] = jnp.sum(cen * cen, axis=0, keepdims=True)

        avg = avg_s[...]
        thr = thr_ref[0]
        keys = jax.lax.bitcast_convert_type(avg, jnp.int32)
        hard_count = jnp.sum((avg > thr).astype(jnp.int32))
        k_eff = jnp.where(hard_count == 0, K_FORCE, K_CAP)

        def bs(_, carry):
            lo, hi = carry
            mid = jax.lax.shift_right_arithmetic(lo + hi, 1)
            cnt = jnp.sum((keys > mid).astype(jnp.int32))
            pred = cnt >= k_eff
            return (jnp.where(pred, mid, lo), jnp.where(pred, hi, mid))

        lo, hi = jax.lax.fori_loop(
            0, 31, bs, (jnp.int32(-1), jnp.int32(0x40000000)))
        use_thr = (hard_count > 0) & (hard_count <= K_CAP)
        sel = jnp.where(use_thr, (avg > thr).astype(jnp.float32),
                        (keys >= hi).astype(jnp.float32))
        ms_ref[...] = avg * sel

    @pl.when(p == 1)
    def _phase1():
        c2 = c2_s[...]
        cen = cen_s[...]
        for i in range(K):
            e = e_refs[i][0]
            prod = jax.lax.dot_general(
                e, cen, (((1,), (0,)), ((), ())),
                preferred_element_type=jnp.float32)
            x2 = jnp.sum(e * e, axis=1, keepdims=True)
            d2 = jnp.maximum(x2 + c2 - 2.0 * prod, 0.0) + 1e-12
            o_ref[i, 0] = d2 * jax.lax.rsqrt(d2)


def _specs():
    e_specs = [
        pl.BlockSpec((1, CH, D), (lambda p, j, i=i: (i, j, 0)))
        for i in range(K)
    ]
    lab_spec = pl.BlockSpec(
        (K, 1, CH, 1), lambda p, j: (0, jnp.where(p == 0, j, 0), 0, 0))
    o_spec = pl.BlockSpec(
        (K, 1, CH, 8), lambda p, j: (0, jnp.where(p == 0, 0, j), 0, 0))
    return dict(
        grid=(2, NB),
        in_specs=(
            [pl.BlockSpec(memory_space=pltpu.SMEM)]
            + [pl.BlockSpec((RROWS, 128), lambda p, j: (0, 0))] * 3
            + [lab_spec] + e_specs
        ),
        out_specs=[pl.BlockSpec((RROWS, 128), lambda p, j: (0, 0)), o_spec],
        out_shape=[
            jax.ShapeDtypeStruct((RROWS, 128), jnp.float32),
            jax.ShapeDtypeStruct((K, NB, CH, 8), jnp.float32),
        ],
        scratch_shapes=[
            pltpu.VMEM((RROWS, 128), jnp.float32),
            pltpu.VMEM((8, D), jnp.float32),
            pltpu.VMEM((1, 8), jnp.float32),
            pltpu.VMEM((D, 8), jnp.float32),
            pltpu.VMEM((1, 8), jnp.float32),
        ],
        compiler_params=pltpu.CompilerParams(
            dimension_semantics=("arbitrary", "arbitrary")),
    )


@functools.partial(jax.jit, static_argnames=("interpret",))
def _run(thr, u0, u1, u2, lab8, e8, interpret=False):
    ms, dist4 = pl.pallas_call(
        _body, **_specs(), interpret=interpret,
    )(thr, u0, u1, u2, lab8, *([e8] * K))
    return ms, dist4


def kernel(uncertainty, embeddings, labels, epoch, max_epochs):
    progress = jnp.minimum(epoch / jnp.maximum(max_epochs - 1, 1), 1.0)
    u_thr = 0.4 + progress * (0.3 - 0.4)
    c_thr = 0.3 + progress * (0.6 - 0.3)
    thr = jnp.minimum(u_thr, -jnp.log(c_thr)).astype(jnp.float32)
    thr = jnp.reshape(thr, (1,))

    u0 = uncertainty[:, 0].reshape(RROWS, 128)
    u1 = uncertainty[:, 1].reshape(RROWS, 128)
    u2 = uncertainty[:, 2].reshape(RROWS, 128)
    lab8 = labels.reshape(K, NB, CH, 1)
    e8 = embeddings.reshape(K, PROWS, D)

    ms, dist4 = _run(thr, u0, u1, u2, lab8, e8)
    dist = dist4.reshape(N, 8)
    return jnp.concatenate([ms.reshape(N, 1), dist[:, :3]], axis=1)


# speedup vs baseline: 1.5873x; 1.4830x over previous
import functools

import jax
import jax.numpy as jnp
from jax.experimental import pallas as pl
from jax.experimental.pallas import tpu as pltpu

N = 65536
D = 256
K = 8
PROWS = N // K
BLK = 8192
NB = N // BLK
CH = BLK // K
RROWS = N // 128
PAVG = PROWS // 128
TAVG = CH // 128
K_FORCE = max(1, int(N * 0.1))
K_CAP = int(N * 0.5)


def _body(*refs):
    thr_ref = refs[0]
    u0_ref, u1_ref, u2_ref = refs[1:4]
    lab_ref = refs[4]
    e_refs = refs[5:5 + K]
    ms_ref = refs[5 + K]
    o_ref = refs[6 + K]
    avg_s, acc_s, cnt_s, cen_s, c2_s = refs[7 + K:]

    p = pl.program_id(0)
    j = pl.program_id(1)

    @pl.when((p == 0) & (j == 0))
    def _init():
        acc_s[...] = jnp.zeros_like(acc_s)
        cnt_s[...] = jnp.zeros_like(cnt_s)

    @pl.when(p == 0)
    def _phase0():
        for i in range(K):
            sl = pl.ds(i * PAVG + j * TAVG, TAVG)
            avg_s[sl, :] = (u0_ref[sl, :] + u1_ref[sl, :] + u2_ref[sl, :]) / 3.0

            e = e_refs[i][0]
            lane = lab_ref[0][:, i * CH:(i + 1) * CH]
            ohfT = (lane == jax.lax.broadcasted_iota(
                jnp.int32, (8, CH), 0)).astype(jnp.float32)
            acc_s[...] += jax.lax.dot_general(
                ohfT, e, (((1,), (0,)), ((), ())),
                preferred_element_type=jnp.float32)
            cnt_s[...] += jnp.sum(ohfT, axis=1, keepdims=True)

    @pl.when((p == 0) & (j == NB - 1))
    def _finalize():
        invc = 1.0 / jnp.maximum(jnp.transpose(cnt_s[...]), 1.0)
        cen = jnp.transpose(acc_s[...]) * invc
        cen_s[...] = cen
        c2_s[...] = jnp.sum(cen * cen, axis=0, keepdims=True)

        avg = avg_s[...]
        thr = thr_ref[0]
        keys = jax.lax.bitcast_convert_type(avg, jnp.int32)
        hard_count = jnp.sum((avg > thr).astype(jnp.int32))
        k_eff = jnp.where(hard_count == 0, K_FORCE, K_CAP)

        def bs(_, carry):
            lo, hi = carry
            mid = jax.lax.shift_right_arithmetic(lo + hi, 1)
            cnt = jnp.sum((keys > mid).astype(jnp.int32))
            pred = cnt >= k_eff
            return (jnp.where(pred, mid, lo), jnp.where(pred, hi, mid))

        lo, hi = jax.lax.fori_loop(
            0, 31, bs, (jnp.int32(-1), jnp.int32(0x40000000)))
        use_thr = (hard_count > 0) & (hard_count <= K_CAP)
        sel = jnp.where(use_thr, (avg > thr).astype(jnp.float32),
                        (keys >= hi).astype(jnp.float32))
        ms_ref[...] = avg * sel

    @pl.when(p == 1)
    def _phase1():
        c2 = c2_s[...]
        cen = cen_s[...]
        for i in range(K):
            e = e_refs[i][0]
            prod = jax.lax.dot_general(
                e, cen, (((1,), (0,)), ((), ())),
                preferred_element_type=jnp.float32)
            x2 = jnp.sum(e * e, axis=1, keepdims=True)
            d2 = jnp.maximum(x2 + c2 - 2.0 * prod, 0.0) + 1e-12
            o_ref[i, 0] = d2 * jax.lax.rsqrt(d2)


def _specs():
    e_specs = [
        pl.BlockSpec((1, CH, D), (lambda p, j, i=i: (i, j, 0)))
        for i in range(K)
    ]
    lab_spec = pl.BlockSpec(
        (1, 1, BLK), lambda p, j: (jnp.where(p == 0, j, 0), 0, 0))
    o_spec = pl.BlockSpec(
        (K, 1, CH, 8), lambda p, j: (0, jnp.where(p == 0, 0, j), 0, 0))
    return dict(
        grid=(2, NB),
        in_specs=(
            [pl.BlockSpec(memory_space=pltpu.SMEM)]
            + [pl.BlockSpec((RROWS, 128), lambda p, j: (0, 0))] * 3
            + [lab_spec] + e_specs
        ),
        out_specs=[pl.BlockSpec((RROWS, 128), lambda p, j: (0, 0)), o_spec],
        out_shape=[
            jax.ShapeDtypeStruct((RROWS, 128), jnp.float32),
            jax.ShapeDtypeStruct((K, NB, CH, 8), jnp.float32),
        ],
        scratch_shapes=[
            pltpu.VMEM((RROWS, 128), jnp.float32),
            pltpu.VMEM((8, D), jnp.float32),
            pltpu.VMEM((8, 1), jnp.float32),
            pltpu.VMEM((D, 8), jnp.float32),
            pltpu.VMEM((1, 8), jnp.float32),
        ],
        compiler_params=pltpu.CompilerParams(
            dimension_semantics=("arbitrary", "arbitrary")),
    )


@functools.partial(jax.jit, static_argnames=("interpret",))
def _run(thr, u0, u1, u2, lab8, e8, interpret=False):
    ms, dist4 = pl.pallas_call(
        _body, **_specs(), interpret=interpret,
    )(thr, u0, u1, u2, lab8, *([e8] * K))
    return ms, dist4


def kernel(uncertainty, embeddings, labels, epoch, max_epochs):
    progress = jnp.minimum(epoch / jnp.maximum(max_epochs - 1, 1), 1.0)
    u_thr = 0.4 + progress * (0.3 - 0.4)
    c_thr = 0.3 + progress * (0.6 - 0.3)
    thr = jnp.minimum(u_thr, -jnp.log(c_thr)).astype(jnp.float32)
    thr = jnp.reshape(thr, (1,))

    u0 = uncertainty[:, 0].reshape(RROWS, 128)
    u1 = uncertainty[:, 1].reshape(RROWS, 128)
    u2 = uncertainty[:, 2].reshape(RROWS, 128)
    lab8 = labels.reshape(K, NB, CH).transpose(1, 0, 2).reshape(NB, 1, BLK)
    e8 = embeddings.reshape(K, PROWS, D)

    ms, dist4 = _run(thr, u0, u1, u2, lab8, e8)
    dist = dist4.reshape(N, 8)
    return jnp.concatenate([ms.reshape(N, 1), dist[:, :3]], axis=1)


# speedup vs baseline: 1.7303x; 1.0901x over previous
import functools

import jax
import jax.numpy as jnp
from jax.experimental import pallas as pl
from jax.experimental.pallas import tpu as pltpu

N = 65536
D = 256
K = 8
PROWS = N // K
BLK = 4096
NB = N // BLK
CH = BLK // K
RROWS = N // 128
PAVG = PROWS // 128
TAVG = CH // 128
K_FORCE = max(1, int(N * 0.1))
K_CAP = int(N * 0.5)


def _body(*refs):
    thr_ref = refs[0]
    u0_ref, u1_ref, u2_ref = refs[1:4]
    lab_ref = refs[4]
    e_refs = refs[5:5 + K]
    ms_ref = refs[5 + K]
    o_ref = refs[6 + K]
    avg_s, acc_s, cnt_s, cen_s, c2_s, ebf_s = refs[7 + K:]

    p = pl.program_id(0)
    j = pl.program_id(1)

    @pl.when((p == 0) & (j == 0))
    def _init():
        acc_s[...] = jnp.zeros_like(acc_s)
        cnt_s[...] = jnp.zeros_like(cnt_s)

    @pl.when(p == 0)
    def _phase0():
        for i in range(K):
            sl = pl.ds(i * PAVG + j * TAVG, TAVG)
            avg_s[sl, :] = (u0_ref[sl, :] + u1_ref[sl, :] + u2_ref[sl, :]) / 3.0

            e = e_refs[i][0]
            lane = lab_ref[0][:, i * CH:(i + 1) * CH]
            ohfT = (lane == jax.lax.broadcasted_iota(
                jnp.int32, (8, CH), 0)).astype(jnp.float32)
            acc_s[...] += jax.lax.dot_general(
                ohfT, e, (((1,), (0,)), ((), ())),
                preferred_element_type=jnp.float32)
            cnt_s[...] += jnp.sum(ohfT, axis=1, keepdims=True)
            ebf_s[pl.ds(i * PROWS + j * CH, CH), :] = e.astype(jnp.bfloat16)

    @pl.when((p == 0) & (j == NB - 1))
    def _finalize():
        invc = 1.0 / jnp.maximum(jnp.transpose(cnt_s[...]), 1.0)
        cen = jnp.transpose(acc_s[...]) * invc
        cen_s[...] = cen
        c2_s[...] = jnp.sum(cen * cen, axis=0, keepdims=True)

        avg = avg_s[...]
        thr = thr_ref[0]
        keys = jax.lax.bitcast_convert_type(avg, jnp.int32)
        hard_count = jnp.sum((avg > thr).astype(jnp.int32))
        k_eff = jnp.where(hard_count == 0, K_FORCE, K_CAP)

        def bs(_, carry):
            lo, hi = carry
            mid = jax.lax.shift_right_arithmetic(lo + hi, 1)
            cnt = jnp.sum((keys > mid).astype(jnp.int32))
            pred = cnt >= k_eff
            return (jnp.where(pred, mid, lo), jnp.where(pred, hi, mid))

        lo, hi = jax.lax.fori_loop(
            0, 31, bs, (jnp.int32(-1), jnp.int32(0x40000000)))
        use_thr = (hard_count > 0) & (hard_count <= K_CAP)
        sel = jnp.where(use_thr, (avg > thr).astype(jnp.float32),
                        (keys >= hi).astype(jnp.float32))
        ms_ref[...] = avg * sel

    @pl.when(p == 1)
    def _phase1():
        c2 = c2_s[...]
        cen = cen_s[...]
        for i in range(K):
            eb = ebf_s[pl.ds(i * PROWS + j * CH, CH), :]
            prod = jax.lax.dot_general(
                eb, cen, (((1,), (0,)), ((), ())),
                preferred_element_type=jnp.float32)
            ef = eb.astype(jnp.float32)
            x2 = jnp.sum(ef * ef, axis=1, keepdims=True)
            d2 = jnp.maximum(x2 + c2 - 2.0 * prod, 0.0) + 1e-12
            o_ref[i, 0] = d2 * jax.lax.rsqrt(d2)


def _specs():
    e_specs = [
        pl.BlockSpec((1, CH, D),
                     (lambda p, j, i=i: (i, jnp.where(p == 0, j, 0), 0)))
        for i in range(K)
    ]
    lab_spec = pl.BlockSpec(
        (1, 1, BLK), lambda p, j: (jnp.where(p == 0, j, 0), 0, 0))
    o_spec = pl.BlockSpec(
        (K, 1, CH, 8), lambda p, j: (0, jnp.where(p == 0, 0, j), 0, 0))
    return dict(
        grid=(2, NB),
        in_specs=(
            [pl.BlockSpec(memory_space=pltpu.SMEM)]
            + [pl.BlockSpec((RROWS, 128), lambda p, j: (0, 0))] * 3
            + [lab_spec] + e_specs
        ),
        out_specs=[pl.BlockSpec((RROWS, 128), lambda p, j: (0, 0)), o_spec],
        out_shape=[
            jax.ShapeDtypeStruct((RROWS, 128), jnp.float32),
            jax.ShapeDtypeStruct((K, NB, CH, 8), jnp.float32),
        ],
        scratch_shapes=[
            pltpu.VMEM((RROWS, 128), jnp.float32),
            pltpu.VMEM((8, D), jnp.float32),
            pltpu.VMEM((8, 1), jnp.float32),
            pltpu.VMEM((D, 8), jnp.float32),
            pltpu.VMEM((1, 8), jnp.float32),
            pltpu.VMEM((N, D), jnp.bfloat16),
        ],
        compiler_params=pltpu.CompilerParams(
            dimension_semantics=("arbitrary", "arbitrary")),
    )


@functools.partial(jax.jit, static_argnames=("interpret",))
def _run(thr, u0, u1, u2, lab8, e8, interpret=False):
    ms, dist4 = pl.pallas_call(
        _body, **_specs(), interpret=interpret,
    )(thr, u0, u1, u2, lab8, *([e8] * K))
    return ms, dist4


def kernel(uncertainty, embeddings, labels, epoch, max_epochs):
    progress = jnp.minimum(epoch / jnp.maximum(max_epochs - 1, 1), 1.0)
    u_thr = 0.4 + progress * (0.3 - 0.4)
    c_thr = 0.3 + progress * (0.6 - 0.3)
    thr = jnp.minimum(u_thr, -jnp.log(c_thr)).astype(jnp.float32)
    thr = jnp.reshape(thr, (1,))

    u0 = uncertainty[:, 0].reshape(RROWS, 128)
    u1 = uncertainty[:, 1].reshape(RROWS, 128)
    u2 = uncertainty[:, 2].reshape(RROWS, 128)
    lab8 = labels.reshape(K, NB, CH).transpose(1, 0, 2).reshape(NB, 1, BLK)
    e8 = embeddings.reshape(K, PROWS, D)

    ms, dist4 = _run(thr, u0, u1, u2, lab8, e8)
    dist = dist4.reshape(N, 8)
    return jnp.concatenate([ms.reshape(N, 1), dist[:, :3]], axis=1)
